# baseline (speedup 1.0000x reference)
import functools

import jax
import jax.numpy as jnp
from jax import lax
from jax.experimental import pallas as pl
from jax.experimental.pallas import tpu as pltpu
from jax.experimental.pallas import tpu_sc as plsc

_SOFT_DIM = 64


def kernel(z):
    B, _, N = z.shape
    P = B * N
    info = plsc.get_sparse_core_info()
    NW = info.num_cores * info.num_subcores
    CH = P // NW
    zf = z.reshape(P)

    mesh = plsc.VectorSubcoreMesh(core_axis_name="c", subcore_axis_name="s")

    @functools.partial(
        pl.kernel,
        mesh=mesh,
        out_type=jax.ShapeDtypeStruct((B * _SOFT_DIM * N,), jnp.float32),
        scratch_types=[
            pltpu.VMEM((CH,), jnp.float32),
            pltpu.VMEM((CH,), jnp.float32),
        ],
    )
    def sc_kernel(z_hbm, out_hbm, zv, rowv):
        c = lax.axis_index("c")
        s = lax.axis_index("s")
        wid = s * info.num_cores + c
        base = wid * CH
        b = base // N
        n0 = base % N
        pltpu.sync_copy(z_hbm.at[pl.ds(base, CH)], zv)

        def norm_body(i, carry):
            x = zv[pl.ds(i * 16, 16)]
            zv[pl.ds(i * 16, 16)] = (jnp.clip(x, -1.0, 1.0) + 1.0) * (
                0.5 * (_SOFT_DIM - 1)
            )
            return carry

        lax.fori_loop(0, CH // 16, norm_body, 0)

        def d_body(d, carry):
            df = d.astype(jnp.float32)

            def c_body(i, inner):
                zn = zv[pl.ds(i * 16, 16)]
                rowv[pl.ds(i * 16, 16)] = 1.0 - jnp.abs(
                    jnp.clip(zn - df, -1.0, 1.0)
                )
                return inner

            lax.fori_loop(0, CH // 16, c_body, 0)
            dst = (b * _SOFT_DIM + d) * N + n0
            pltpu.sync_copy(rowv, out_hbm.at[pl.ds(dst, CH)])
            return carry

        lax.fori_loop(0, _SOFT_DIM, d_body, 0)

    out = sc_kernel(zf)
    return out.reshape(B, _SOFT_DIM, N)


# speedup vs baseline: 8.5452x; 8.5452x over previous
import jax
import jax.numpy as jnp
from jax.experimental import pallas as pl

_SOFT_DIM = 64
_TILE_N = 32768
_TILE_B = 2


def _triangle_kernel(z_ref, out_ref):
    z = z_ref[...]
    z_norm = (jnp.clip(z, -1.0, 1.0) + 1.0) * (0.5 * (_SOFT_DIM - 1))
    d = jax.lax.broadcasted_iota(jnp.int32, out_ref.shape, 1).astype(jnp.float32)
    out_ref[...] = jnp.maximum(1.0 - jnp.abs(z_norm - d), 0.0)


def kernel(z):
    B, _, N = z.shape
    tile = _TILE_N if N % _TILE_N == 0 else N
    tile_b = _TILE_B if B % _TILE_B == 0 else 1
    return pl.pallas_call(
        _triangle_kernel,
        grid=(B // tile_b, N // tile),
        in_specs=[pl.BlockSpec((tile_b, 1, tile), lambda b, n: (b, 0, n))],
        out_specs=pl.BlockSpec((tile_b, _SOFT_DIM, tile), lambda b, n: (b, 0, n)),
        out_shape=jax.ShapeDtypeStruct((B, _SOFT_DIM, N), z.dtype),
    )(z)


# speedup vs baseline: 8.8788x; 1.0390x over previous
import jax
import jax.numpy as jnp
from jax.experimental import pallas as pl

_SOFT_DIM = 64
_TILE_N = 32768


def _triangle_kernel(z_ref, out_ref):
    z = z_ref[...]
    z_norm = (jnp.clip(z, -1.0, 1.0) + 1.0) * (0.5 * (_SOFT_DIM - 1))
    d = jax.lax.broadcasted_iota(jnp.int32, out_ref.shape, 1).astype(jnp.float32)
    out_ref[...] = jnp.maximum(1.0 - jnp.abs(z_norm - d), 0.0)


def kernel(z):
    B, _, N = z.shape
    tile = _TILE_N if N % _TILE_N == 0 else N
    return pl.pallas_call(
        _triangle_kernel,
        grid=(B, N // tile),
        in_specs=[pl.BlockSpec((1, 1, tile), lambda b, n: (b, 0, n))],
        out_specs=pl.BlockSpec((1, _SOFT_DIM, tile), lambda b, n: (b, 0, n)),
        out_shape=jax.ShapeDtypeStruct((B, _SOFT_DIM, N), z.dtype),
    )(z)
